# Initial kernel scaffold; baseline (speedup 1.0000x reference)
import jax
import jax.numpy as jnp
from jax.experimental import pallas as pl


def kernel(token, codebook):
    raise NotImplementedError("write your pallas kernel here")


# speedup vs baseline: 1.2244x; 1.2244x over previous
import functools

import jax
import jax.numpy as jnp
from jax import lax
from jax.experimental import pallas as pl
from jax.experimental.pallas import tpu as pltpu
from jax.experimental.pallas import tpu_sc as plsc

_B, _N, _D, _K = 16, 1024, 768, 8192
_T = _B * _N
_TM = 256
_TK = 512
_SUB = _K // _TK
_TT = _T // _TM


def _argmin_body(x_ref, cb_ref, x2_ref, c2_ref, idx_ref, loss_ref, loss_scr):
    t = pl.program_id(0)

    x = x_ref[...].astype(jnp.bfloat16)
    x2 = x2_ref[...]

    rv = jnp.full((_TM,), jnp.inf, jnp.float32)
    ri = jnp.zeros((_TM,), jnp.int32)
    iota = lax.broadcasted_iota(jnp.int32, (_TM, _TK), 1)
    for s in range(_SUB):
        c = cb_ref[s * _TK:(s + 1) * _TK, :].astype(jnp.bfloat16)
        dots = lax.dot_general(x, c, (((1,), (1,)), ((), ())),
                               preferred_element_type=jnp.float32)
        dist = (x2 - 2.0 * dots) + c2_ref[:, s * _TK:(s + 1) * _TK]
        lmin = jnp.min(dist, axis=1)
        cand = jnp.where(dist == lmin[:, None], iota, _K)
        li = jnp.min(cand, axis=1)
        better = lmin < rv
        rv = jnp.where(better, lmin, rv)
        ri = jnp.where(better, li + s * _TK, ri)

    idx_ref[0, 0, :] = ri

    part = jnp.sum(rv)
    prev = jnp.where(t == 0, 0.0, loss_scr[0, 0])
    loss_scr[0, 0] = prev + part

    @pl.when(t == _TT - 1)
    def _():
        loss_ref[0, 0] = loss_scr[0, 0] / jnp.float32(_T * _D)


def _assign(x_flat, codebook, x2_col, c2_row):
    return pl.pallas_call(
        _argmin_body,
        grid=(_TT,),
        in_specs=[
            pl.BlockSpec((_TM, _D), lambda t: (t, 0)),
            pl.BlockSpec((_K, _D), lambda t: (0, 0)),
            pl.BlockSpec((_TM, 1), lambda t: (t, 0)),
            pl.BlockSpec((1, _K), lambda t: (0, 0)),
        ],
        out_specs=[
            pl.BlockSpec((1, 1, _TM), lambda t: (t, 0, 0)),
            pl.BlockSpec(block_shape=(1, 1), index_map=lambda t: (0, 0),
                         memory_space=pltpu.SMEM),
        ],
        out_shape=[
            jax.ShapeDtypeStruct((_TT, 1, _TM), jnp.int32),
            jax.ShapeDtypeStruct((1, 1), jnp.float32),
        ],
        scratch_shapes=[
            pltpu.SMEM((1, 1), jnp.float32),
        ],
    )(x_flat, codebook, x2_col, c2_row)


_NC, _NS = 2, 16
_NW = _NC * _NS
_RPW = _T // _NW
_CHUNK = 128
_NCHUNK = _RPW // _CHUNK


def _gather_body(cb_hbm, idx_hbm, out_hbm, idx_v, rows_v, sem):
    wid = lax.axis_index("s") * _NC + lax.axis_index("c")
    base = wid * _RPW

    def chunk(i, carry):
        off = base + i * _CHUNK
        pltpu.sync_copy(idx_hbm.at[pl.ds(off, _CHUNK)], idx_v)
        pltpu.async_copy(cb_hbm.at[idx_v], rows_v, sem).wait()
        pltpu.sync_copy(rows_v, out_hbm.at[pl.ds(off, _CHUNK)])
        return carry

    lax.fori_loop(0, _NCHUNK, chunk, 0)


@functools.cache
def _gather_kernel():
    return pl.kernel(
        _gather_body,
        out_type=jax.ShapeDtypeStruct((_T, _D), jnp.float32),
        mesh=plsc.VectorSubcoreMesh(core_axis_name="c", subcore_axis_name="s",
                                    num_cores=_NC),
        scratch_types=[
            pltpu.VMEM((_CHUNK,), jnp.int32),
            pltpu.VMEM((_CHUNK, _D), jnp.float32),
            pltpu.SemaphoreType.DMA,
        ],
    )


def kernel(token, codebook):
    x2 = jnp.sum(token * token, axis=-1, keepdims=True)
    c2 = jnp.sum(codebook * codebook, axis=-1)
    x_flat = token.reshape(_T, _D)
    idx3, loss2 = _assign(x_flat, codebook,
                          x2.reshape(_T, 1), c2.reshape(1, _K))
    idx_flat = idx3.reshape(_T)
    quant = _gather_kernel()(codebook, idx_flat).reshape(_B, _N, _D)
    indices = idx_flat.reshape(_B, _N)
    return quant, indices, token, loss2[0, 0]


# speedup vs baseline: 1.2491x; 1.0202x over previous
import functools

import jax
import jax.numpy as jnp
from jax import lax
from jax.experimental import pallas as pl
from jax.experimental.pallas import tpu as pltpu
from jax.experimental.pallas import tpu_sc as plsc

_B, _N, _D, _K = 16, 1024, 768, 8192
_T = _B * _N
_TM = 256
_TK = 512
_SUB = _K // _TK
_TT = _T // _TM


def _argmin_body(x_ref, cb2_ref, x2_ref, c2_ref, idx_ref, loss_ref, loss_scr):
    t = pl.program_id(0)

    x = x_ref[...].astype(jnp.bfloat16)
    x2 = x2_ref[...]

    rv = jnp.full((_TM,), jnp.inf, jnp.float32)
    ri = jnp.zeros((_TM,), jnp.int32)
    iota = lax.broadcasted_iota(jnp.int32, (_TM, _TK), 1)
    for s in range(_SUB):
        c = cb2_ref[s * _TK:(s + 1) * _TK, :].astype(jnp.bfloat16)
        dots2 = lax.dot_general(x, c, (((1,), (1,)), ((), ())),
                                preferred_element_type=jnp.float32)
        dist = (x2 - dots2) + c2_ref[:, s * _TK:(s + 1) * _TK]
        lmin = jnp.min(dist, axis=1)
        cand = jnp.where(dist == lmin[:, None], iota, _K)
        li = jnp.min(cand, axis=1)
        better = lmin < rv
        rv = jnp.where(better, lmin, rv)
        ri = jnp.where(better, li + s * _TK, ri)

    idx_ref[0, 0, :] = ri

    part = jnp.sum(rv)
    prev = jnp.where(t == 0, 0.0, loss_scr[0, 0])
    loss_scr[0, 0] = prev + part

    @pl.when(t == _TT - 1)
    def _():
        loss_ref[0, 0] = loss_scr[0, 0] / jnp.float32(_T * _D)


def _assign(x_flat, codebook2, x2_col, c2_row):
    return pl.pallas_call(
        _argmin_body,
        grid=(_TT,),
        in_specs=[
            pl.BlockSpec((_TM, _D), lambda t: (t, 0)),
            pl.BlockSpec((_K, _D), lambda t: (0, 0)),
            pl.BlockSpec((_TM, 1), lambda t: (t, 0)),
            pl.BlockSpec((1, _K), lambda t: (0, 0)),
        ],
        out_specs=[
            pl.BlockSpec((1, 1, _TM), lambda t: (t, 0, 0)),
            pl.BlockSpec(block_shape=(1, 1), index_map=lambda t: (0, 0),
                         memory_space=pltpu.SMEM),
        ],
        out_shape=[
            jax.ShapeDtypeStruct((_TT, 1, _TM), jnp.int32),
            jax.ShapeDtypeStruct((1, 1), jnp.float32),
        ],
        scratch_shapes=[
            pltpu.SMEM((1, 1), jnp.float32),
        ],
    )(x_flat, codebook2, x2_col, c2_row)


_NC, _NS = 2, 16
_NW = _NC * _NS
_RPW = _T // _NW
_CHUNK = 128
_NCHUNK = _RPW // _CHUNK


def _gather_body(cb_hbm, idx_hbm, out_hbm, idx_v, rows_v, sem):
    wid = lax.axis_index("s") * _NC + lax.axis_index("c")
    base = wid * _RPW

    def chunk(i, carry):
        off = base + i * _CHUNK
        pltpu.sync_copy(idx_hbm.at[pl.ds(off, _CHUNK)], idx_v)
        pltpu.async_copy(cb_hbm.at[idx_v], rows_v, sem).wait()
        pltpu.sync_copy(rows_v, out_hbm.at[pl.ds(off, _CHUNK)])
        return carry

    lax.fori_loop(0, _NCHUNK, chunk, 0)


@functools.cache
def _gather_kernel():
    return pl.kernel(
        _gather_body,
        out_type=jax.ShapeDtypeStruct((_T, _D), jnp.float32),
        mesh=plsc.VectorSubcoreMesh(core_axis_name="c", subcore_axis_name="s",
                                    num_cores=_NC),
        scratch_types=[
            pltpu.VMEM((_CHUNK,), jnp.int32),
            pltpu.VMEM((_CHUNK, _D), jnp.float32),
            pltpu.SemaphoreType.DMA,
        ],
    )


def kernel(token, codebook):
    x2 = jnp.sum(token * token, axis=-1, keepdims=True)
    c2 = jnp.sum(codebook * codebook, axis=-1)
    x_flat = token.reshape(_T, _D)
    idx3, loss2 = _assign(x_flat, codebook * 2.0,
                          x2.reshape(_T, 1), c2.reshape(1, _K))
    idx_flat = idx3.reshape(_T)
    quant = _gather_kernel()(codebook, idx_flat).reshape(_B, _N, _D)
    indices = idx_flat.reshape(_B, _N)
    return quant, indices, token, loss2[0, 0]
